# baseline (speedup 1.0000x reference)
import jax
import jax.numpy as jnp
from jax.experimental import pallas as pl
from jax.experimental.pallas import tpu as pltpu

C, H, W, S = 96, 512, 512, 128
R_BLK = 1024


def _flip_body(sites_ref, x_ref, o_ref, a_ref, b_ref):
    @pl.when(pl.program_id(0) == 0)
    def _build_mask():
        sites = sites_ref[...]
        col = jax.lax.broadcasted_iota(jnp.int32, (S, W), 1)
        m = jnp.any(col == sites, axis=0, keepdims=True)
        mf = m.astype(jnp.float32)
        a_ref[...] = jnp.broadcast_to(1.0 - 2.0 * mf, (8, W))
        b_ref[...] = jnp.broadcast_to(mf, (8, W))

    xv = x_ref[...]
    a = a_ref[...]
    b = b_ref[...]
    rep = R_BLK // 8
    o_ref[...] = xv * jnp.tile(a, (rep, 1)) + jnp.tile(b, (rep, 1))


def kernel(x, mask_sites):
    x2 = x.reshape(C * H, W)
    sites2 = mask_sites.reshape(S, 1)
    out = pl.pallas_call(
        _flip_body,
        grid=((C * H) // R_BLK,),
        in_specs=[
            pl.BlockSpec((S, 1), lambda i: (0, 0)),
            pl.BlockSpec((R_BLK, W), lambda i: (i, 0)),
        ],
        out_specs=pl.BlockSpec((R_BLK, W), lambda i: (i, 0)),
        out_shape=jax.ShapeDtypeStruct((C * H, W), jnp.float32),
        scratch_shapes=[
            pltpu.VMEM((8, W), jnp.float32),
            pltpu.VMEM((8, W), jnp.float32),
        ],
    )(sites2, x2)
    return out.reshape(C, H, W)


# speedup vs baseline: 1.1292x; 1.1292x over previous
import jax
import jax.numpy as jnp
from jax.experimental import pallas as pl
from jax.experimental.pallas import tpu as pltpu

C, H, W, S = 96, 512, 512, 128
R_BLK = 4096


def _flip_body(sites_ref, x_ref, o_ref, a_ref, b_ref):
    @pl.when(pl.program_id(0) == 0)
    def _build_mask():
        sites = sites_ref[...]
        col = jax.lax.broadcasted_iota(jnp.int32, (S, W), 1)
        m = jnp.any(col == sites, axis=0, keepdims=True)
        mf = m.astype(jnp.float32)
        a_ref[...] = jnp.broadcast_to(1.0 - 2.0 * mf, (8, W))
        b_ref[...] = jnp.broadcast_to(mf, (8, W))

    xv = x_ref[...]
    a = a_ref[...]
    b = b_ref[...]
    rep = R_BLK // 8
    o_ref[...] = xv * jnp.tile(a, (rep, 1)) + jnp.tile(b, (rep, 1))


def kernel(x, mask_sites):
    x2 = x.reshape(C * H, W)
    sites2 = mask_sites.reshape(S, 1)
    out = pl.pallas_call(
        _flip_body,
        grid=((C * H) // R_BLK,),
        in_specs=[
            pl.BlockSpec((S, 1), lambda i: (0, 0)),
            pl.BlockSpec((R_BLK, W), lambda i: (i, 0)),
        ],
        out_specs=pl.BlockSpec((R_BLK, W), lambda i: (i, 0)),
        out_shape=jax.ShapeDtypeStruct((C * H, W), jnp.float32),
        scratch_shapes=[
            pltpu.VMEM((8, W), jnp.float32),
            pltpu.VMEM((8, W), jnp.float32),
        ],
    )(sites2, x2)
    return out.reshape(C, H, W)


# speedup vs baseline: 1.1311x; 1.0016x over previous
import jax
import jax.numpy as jnp
from jax.experimental import pallas as pl
from jax.experimental.pallas import tpu as pltpu

C, H, W, S = 96, 512, 512, 128
R_BLK = 6144


def _flip_body(sites_ref, x_ref, o_ref, a_ref, b_ref):
    @pl.when(pl.program_id(0) == 0)
    def _build_mask():
        sites = sites_ref[...]
        col = jax.lax.broadcasted_iota(jnp.int32, (S, W), 1)
        m = jnp.any(col == sites, axis=0, keepdims=True)
        mf = m.astype(jnp.float32)
        a_ref[...] = jnp.broadcast_to(1.0 - 2.0 * mf, (8, W))
        b_ref[...] = jnp.broadcast_to(mf, (8, W))

    xv = x_ref[...]
    a = a_ref[...]
    b = b_ref[...]
    rep = R_BLK // 8
    o_ref[...] = xv * jnp.tile(a, (rep, 1)) + jnp.tile(b, (rep, 1))


def kernel(x, mask_sites):
    x2 = x.reshape(C * H, W)
    sites2 = mask_sites.reshape(S, 1)
    out = pl.pallas_call(
        _flip_body,
        grid=((C * H) // R_BLK,),
        in_specs=[
            pl.BlockSpec((S, 1), lambda i: (0, 0)),
            pl.BlockSpec((R_BLK, W), lambda i: (i, 0)),
        ],
        out_specs=pl.BlockSpec((R_BLK, W), lambda i: (i, 0)),
        out_shape=jax.ShapeDtypeStruct((C * H, W), jnp.float32),
        scratch_shapes=[
            pltpu.VMEM((8, W), jnp.float32),
            pltpu.VMEM((8, W), jnp.float32),
        ],
    )(sites2, x2)
    return out.reshape(C, H, W)
